# baseline (speedup 1.0000x reference)
import jax
import jax.numpy as jnp
from jax import lax
from jax.experimental import pallas as pl
from jax.experimental.pallas import tpu as pltpu
from jax.experimental.pallas import tpu_sc as plsc

N_NODES = 10000
NPAD = 10240
N_EDGES = 320000
D = 128
ALPHA = 0.2
NC, NS, L = 2, 16, 16
NW = NC * NS
EB = 128
KS = 8
NI = 80
NSC = NI // KS
E_PAD = NI * NW * EB
NB_PAD = E_PAD // EB
TILE_ROWS = NPAD // NS
ROW_CHUNK = 128
N_CHUNKS = TILE_ROWS // ROW_CHUNK


def _scores_body(head_ref, tail_ref, attn_ref, ah_ref, at_ref):
    aw = attn_ref[...]
    ah_ref[...] = jnp.sum(head_ref[...] * aw[:, :D], axis=1, keepdims=True)
    at_ref[...] = jnp.sum(tail_ref[...] * aw[:, D:], axis=1, keepdims=True)


def _sc_body(ah_hbm, at_hbm, tv_hbm, src_hbm, dst_hbm,
             hp_out, rs_out,
             ah_tab, at_tab, src_sb, dst_sb, src1, dst1, w_buf, rows,
             hp_acc, rs_acc, sem):
    c = lax.axis_index("c")
    s = lax.axis_index("s")
    wid = s * NC + c

    zero16 = jnp.zeros((L,), jnp.float32)

    def _zbody(r, carry):
        for j in range(D // L):
            rows[r, pl.ds(j * L, L)] = zero16
        return carry

    lax.fori_loop(0, ROW_CHUNK, _zbody, 0)

    tbase = s * TILE_ROWS
    for k in range(N_CHUNKS):
        pltpu.sync_copy(rows, hp_acc.at[pl.ds(tbase + k * ROW_CHUNK, ROW_CHUNK)])
        pltpu.sync_copy(rows.at[0], rs_acc.at[pl.ds(tbase + k * ROW_CHUNK, ROW_CHUNK)])

    pltpu.sync_copy(ah_hbm, ah_tab)
    pltpu.sync_copy(at_hbm, at_tab)

    plsc.subcore_barrier()

    sc0 = wid * NI

    def _sbody(j, carry):
        b0 = pl.multiple_of(sc0 + j * KS, KS)
        pltpu.sync_copy(src_hbm.at[pl.ds(b0, KS)], src_sb)
        pltpu.sync_copy(dst_hbm.at[pl.ds(b0, KS)], dst_sb)

        for u in range(KS):
            for jj in range(EB // L):
                src1[pl.ds(jj * L, L)] = src_sb[u, pl.ds(jj * L, L)]
                dst1[pl.ds(jj * L, L)] = dst_sb[u, pl.ds(jj * L, L)]
            pltpu.async_copy(tv_hbm.at[dst1], rows, sem).wait()
            for jj in range(EB // L):
                si = src1[pl.ds(jj * L, L)]
                di = dst1[pl.ds(jj * L, L)]
                x = plsc.load_gather(ah_tab, [si]) + plsc.load_gather(at_tab, [di])
                x = jnp.clip(x, -10.0, 10.0)
                x = jnp.where(x >= 0.0, x, ALPHA * x)
                w_buf[pl.ds(jj * L, L)] = jnp.exp(x)

            def _mbody(g, mcarry):
                wv = w_buf[pl.ds(g * L, L)]
                for l in range(L):
                    wr = wv[l]
                    r = g * L + l
                    for kk in range(D // L):
                        rows[r, pl.ds(kk * L, L)] = rows[r, pl.ds(kk * L, L)] * wr
                return mcarry

            lax.fori_loop(0, EB // L, _mbody, 0)
            pltpu.sync_copy(rows, hp_acc.at[src1], add=True)
            pltpu.sync_copy(w_buf, rs_acc.at[src1], add=True)
        return carry

    lax.fori_loop(0, NSC, _sbody, 0)

    plsc.subcore_barrier()

    pltpu.sync_copy(hp_acc.at[pl.ds(tbase, TILE_ROWS)],
                    hp_out.at[c, pl.ds(tbase, TILE_ROWS)])
    pltpu.sync_copy(rs_acc.at[pl.ds(tbase, TILE_ROWS)],
                    rs_out.at[c, pl.ds(tbase, TILE_ROWS)])


_CB = 1024


def _combine_body(hp_ref, rs_ref, hp_out_ref, rs_out_ref):
    hp_out_ref[...] = hp_ref[0] + hp_ref[1]
    rs_out_ref[...] = (rs_ref[0] + rs_ref[1])[:, None]


def kernel(head_rep, tail_rep, tail_val, edge_list, rel_list, attn):
    f32 = jnp.float32
    i32 = jnp.int32
    head_p = jnp.pad(head_rep.astype(f32), ((0, NPAD - N_NODES), (0, 0)))
    tail_p = jnp.pad(tail_rep.astype(f32), ((0, NPAD - N_NODES), (0, 0)))
    pad_n = E_PAD - N_EDGES
    src_p = jnp.concatenate([edge_list[0].astype(i32),
                             jnp.full((pad_n,), N_NODES, dtype=i32)])
    dst_p = jnp.concatenate([edge_list[1].astype(i32),
                             jnp.zeros((pad_n,), dtype=i32)])
    src2d = src_p.reshape(NB_PAD, EB)
    dst2d = dst_p.reshape(NB_PAD, EB)

    ah2, at2 = pl.pallas_call(
        _scores_body,
        grid=(NPAD // _CB,),
        in_specs=[
            pl.BlockSpec((_CB, D), lambda i: (i, 0)),
            pl.BlockSpec((_CB, D), lambda i: (i, 0)),
            pl.BlockSpec((1, 2 * D), lambda i: (0, 0)),
        ],
        out_specs=[
            pl.BlockSpec((_CB, 1), lambda i: (i, 0)),
            pl.BlockSpec((_CB, 1), lambda i: (i, 0)),
        ],
        out_shape=[
            jax.ShapeDtypeStruct((NPAD, 1), f32),
            jax.ShapeDtypeStruct((NPAD, 1), f32),
        ],
    )(head_p, tail_p, attn.astype(f32))
    ah = ah2.reshape(NPAD)
    at = at2.reshape(NPAD)

    mesh = plsc.VectorSubcoreMesh(core_axis_name="c", subcore_axis_name="s")
    sc_fn = pl.kernel(
        _sc_body,
        mesh=mesh,
        compiler_params=pltpu.CompilerParams(needs_layout_passes=False),
        out_type=[
            jax.ShapeDtypeStruct((NC, NPAD, D), f32),
            jax.ShapeDtypeStruct((NC, NPAD), f32),
        ],
        scratch_types=[
            pltpu.VMEM((NPAD,), f32),
            pltpu.VMEM((NPAD,), f32),
            pltpu.VMEM((KS, EB), i32),
            pltpu.VMEM((KS, EB), i32),
            pltpu.VMEM((EB,), i32),
            pltpu.VMEM((EB,), i32),
            pltpu.VMEM((EB,), f32),
            pltpu.VMEM((EB, D), f32),
            pltpu.VMEM_SHARED((NPAD, D), f32),
            pltpu.VMEM_SHARED((NPAD,), f32),
            pltpu.SemaphoreType.DMA,
        ],
    )
    hp_part, rs_part = sc_fn(ah, at, tail_val.astype(f32), src2d, dst2d)

    hp, rs = pl.pallas_call(
        _combine_body,
        grid=(NPAD // _CB,),
        in_specs=[
            pl.BlockSpec((NC, _CB, D), lambda i: (0, i, 0)),
            pl.BlockSpec((NC, _CB), lambda i: (0, i)),
        ],
        out_specs=[
            pl.BlockSpec((_CB, D), lambda i: (i, 0)),
            pl.BlockSpec((_CB, 1), lambda i: (i, 0)),
        ],
        out_shape=[
            jax.ShapeDtypeStruct((N_NODES, D), f32),
            jax.ShapeDtypeStruct((N_NODES, 1), f32),
        ],
    )(hp_part, rs_part)

    return (rs, hp)


# speedup vs baseline: 1.0018x; 1.0018x over previous
import jax
import jax.numpy as jnp
from jax import lax
from jax.experimental import pallas as pl
from jax.experimental.pallas import tpu as pltpu
from jax.experimental.pallas import tpu_sc as plsc

N_NODES = 10000
NPAD = 10240
N_EDGES = 320000
D = 128
ALPHA = 0.2
NC, NS, L = 2, 16, 16
NW = NC * NS
EB = 128
KS = 8
NI = 80
NSC = NI // KS
E_PAD = NI * NW * EB
NB_PAD = E_PAD // EB
TILE_ROWS = NPAD // NS
ROW_CHUNK = 128
N_CHUNKS = TILE_ROWS // ROW_CHUNK


def _scores_body(head_ref, tail_ref, attn_ref, ah_ref, at_ref):
    aw = attn_ref[...]
    ah_ref[...] = jnp.sum(head_ref[...] * aw[:, :D], axis=1, keepdims=True)
    at_ref[...] = jnp.sum(tail_ref[...] * aw[:, D:], axis=1, keepdims=True)


def _sc_body(ah_hbm, at_hbm, tv_hbm, src_hbm, dst_hbm,
             hp_out, rs_out,
             ah_tab, at_tab, src_sb, dst_sb, src1, dst1, w_buf, rows,
             hp_acc, rs_acc, sem):
    c = lax.axis_index("c")
    s = lax.axis_index("s")
    wid = s * NC + c

    zero16 = jnp.zeros((L,), jnp.float32)

    def _zbody(r, carry):
        for j in range(D // L):
            rows[r, pl.ds(j * L, L)] = zero16
        return carry

    lax.fori_loop(0, ROW_CHUNK, _zbody, 0)

    tbase = s * TILE_ROWS
    for k in range(N_CHUNKS):
        pltpu.sync_copy(rows, hp_acc.at[pl.ds(tbase + k * ROW_CHUNK, ROW_CHUNK)])
        pltpu.sync_copy(rows.at[0], rs_acc.at[pl.ds(tbase + k * ROW_CHUNK, ROW_CHUNK)])

    pltpu.sync_copy(ah_hbm, ah_tab)
    pltpu.sync_copy(at_hbm, at_tab)

    plsc.subcore_barrier()

    sc0 = wid * NI

    def _sbody(j, carry):
        b0 = pl.multiple_of(sc0 + j * KS, KS)
        pltpu.sync_copy(src_hbm.at[pl.ds(b0, KS)], src_sb)
        pltpu.sync_copy(dst_hbm.at[pl.ds(b0, KS)], dst_sb)

        for u in range(KS):
            for jj in range(EB // L):
                src1[pl.ds(jj * L, L)] = src_sb[u, pl.ds(jj * L, L)]
                dst1[pl.ds(jj * L, L)] = dst_sb[u, pl.ds(jj * L, L)]
            pltpu.async_copy(tv_hbm.at[dst1], rows, sem).wait()
            for jj in range(EB // L):
                si = src1[pl.ds(jj * L, L)]
                di = dst1[pl.ds(jj * L, L)]
                x = plsc.load_gather(ah_tab, [si]) + plsc.load_gather(at_tab, [di])
                x = jnp.clip(x, -10.0, 10.0)
                x = jnp.where(x >= 0.0, x, ALPHA * x)
                w_buf[pl.ds(jj * L, L)] = jnp.exp(x)

            def _mbody(g, mcarry):
                wv = w_buf[pl.ds(g * L, L)]
                for l in range(L):
                    wr = wv[l]
                    r = g * L + l
                    for kk in range(D // L):
                        rows[r, pl.ds(kk * L, L)] = rows[r, pl.ds(kk * L, L)] * wr
                return mcarry

            lax.fori_loop(0, EB // L, _mbody, 0)
            pltpu.sync_copy(rows, hp_acc.at[src1], add=True)
            pltpu.sync_copy(w_buf, rs_acc.at[src1], add=True)
        return carry

    lax.fori_loop(0, NSC, _sbody, 0)

    plsc.subcore_barrier()

    pltpu.sync_copy(hp_acc.at[pl.ds(tbase, TILE_ROWS)],
                    hp_out.at[c, pl.ds(tbase, TILE_ROWS)])
    pltpu.sync_copy(rs_acc.at[pl.ds(tbase, TILE_ROWS)],
                    rs_out.at[c, pl.ds(tbase, TILE_ROWS)])


_CB = 1024


def _combine_body(hp_ref, rs_ref, hp_out_ref, rs_out_ref):
    hp_out_ref[...] = hp_ref[0] + hp_ref[1]
    rs_out_ref[...] = (rs_ref[0] + rs_ref[1])[:, None]


def kernel(head_rep, tail_rep, tail_val, edge_list, rel_list, attn):
    f32 = jnp.float32
    i32 = jnp.int32
    head_p = jnp.pad(head_rep.astype(f32), ((0, NPAD - N_NODES), (0, 0)))
    tail_p = jnp.pad(tail_rep.astype(f32), ((0, NPAD - N_NODES), (0, 0)))
    pad_n = E_PAD - N_EDGES
    pad_src = N_NODES + (jnp.arange(pad_n, dtype=i32) % (NPAD - N_NODES))
    src_p = jnp.concatenate([edge_list[0].astype(i32), pad_src])
    dst_p = jnp.concatenate([edge_list[1].astype(i32),
                             jnp.zeros((pad_n,), dtype=i32)])
    src2d = src_p.reshape(NB_PAD, EB)
    dst2d = dst_p.reshape(NB_PAD, EB)

    ah2, at2 = pl.pallas_call(
        _scores_body,
        grid=(NPAD // _CB,),
        in_specs=[
            pl.BlockSpec((_CB, D), lambda i: (i, 0)),
            pl.BlockSpec((_CB, D), lambda i: (i, 0)),
            pl.BlockSpec((1, 2 * D), lambda i: (0, 0)),
        ],
        out_specs=[
            pl.BlockSpec((_CB, 1), lambda i: (i, 0)),
            pl.BlockSpec((_CB, 1), lambda i: (i, 0)),
        ],
        out_shape=[
            jax.ShapeDtypeStruct((NPAD, 1), f32),
            jax.ShapeDtypeStruct((NPAD, 1), f32),
        ],
    )(head_p, tail_p, attn.astype(f32))
    ah = ah2.reshape(NPAD)
    at = at2.reshape(NPAD)

    mesh = plsc.VectorSubcoreMesh(core_axis_name="c", subcore_axis_name="s")
    sc_fn = pl.kernel(
        _sc_body,
        mesh=mesh,
        compiler_params=pltpu.CompilerParams(needs_layout_passes=False),
        out_type=[
            jax.ShapeDtypeStruct((NC, NPAD, D), f32),
            jax.ShapeDtypeStruct((NC, NPAD), f32),
        ],
        scratch_types=[
            pltpu.VMEM((NPAD,), f32),
            pltpu.VMEM((NPAD,), f32),
            pltpu.VMEM((KS, EB), i32),
            pltpu.VMEM((KS, EB), i32),
            pltpu.VMEM((EB,), i32),
            pltpu.VMEM((EB,), i32),
            pltpu.VMEM((EB,), f32),
            pltpu.VMEM((EB, D), f32),
            pltpu.VMEM_SHARED((NPAD, D), f32),
            pltpu.VMEM_SHARED((NPAD,), f32),
            pltpu.SemaphoreType.DMA,
        ],
    )
    hp_part, rs_part = sc_fn(ah, at, tail_val.astype(f32), src2d, dst2d)

    hp, rs = pl.pallas_call(
        _combine_body,
        grid=(NPAD // _CB,),
        in_specs=[
            pl.BlockSpec((NC, _CB, D), lambda i: (0, i, 0)),
            pl.BlockSpec((NC, _CB), lambda i: (0, i)),
        ],
        out_specs=[
            pl.BlockSpec((_CB, D), lambda i: (i, 0)),
            pl.BlockSpec((_CB, 1), lambda i: (i, 0)),
        ],
        out_shape=[
            jax.ShapeDtypeStruct((N_NODES, D), f32),
            jax.ShapeDtypeStruct((N_NODES, 1), f32),
        ],
    )(hp_part, rs_part)

    return (rs, hp)


# speedup vs baseline: 1.9204x; 1.9170x over previous
import jax
import jax.numpy as jnp
from jax import lax
from jax.experimental import pallas as pl
from jax.experimental.pallas import tpu as pltpu
from jax.experimental.pallas import tpu_sc as plsc

N_NODES = 10000
NPAD = 10240
N_EDGES = 320000
D = 128
ALPHA = 0.2
NC, NS, L = 2, 16, 16
NW = NC * NS
EB = 128
N_BATCHES = N_EDGES // EB
E_PAD = (N_BATCHES + NW) * EB
TILE_ROWS = NPAD // NS
ROW_CHUNK = 128
N_CHUNKS = TILE_ROWS // ROW_CHUNK
FULL_I = N_BATCHES // NW
EXTRA = N_BATCHES - FULL_I * NW


def _scores_body(head_ref, tail_ref, attn_ref, ah_ref, at_ref):
    aw = attn_ref[...]
    ah_ref[...] = jnp.sum(head_ref[...] * aw[:, :D], axis=1, keepdims=True)
    at_ref[...] = jnp.sum(tail_ref[...] * aw[:, D:], axis=1, keepdims=True)


def _sc_body(ah_hbm, at_hbm, tv_hbm, src_hbm, dst_hbm,
             hp_out, rs_out,
             ah_tab, at_tab, src_buf, dst_buf, w_buf, rows,
             hp_acc, rs_acc, sem, sem_is, sem_id):
    c = lax.axis_index("c")
    s = lax.axis_index("s")
    wid = s * NC + c

    zero16 = jnp.zeros((L,), jnp.float32)

    def _zbody(r, carry):
        for j in range(D // L):
            rows[r, pl.ds(j * L, L)] = zero16
        return carry

    lax.fori_loop(0, ROW_CHUNK, _zbody, 0)

    tbase = s * TILE_ROWS
    for k in range(N_CHUNKS):
        pltpu.sync_copy(rows, hp_acc.at[pl.ds(tbase + k * ROW_CHUNK, ROW_CHUNK)])
        pltpu.sync_copy(rows.at[0], rs_acc.at[pl.ds(tbase + k * ROW_CHUNK, ROW_CHUNK)])

    pltpu.sync_copy(ah_hbm, ah_tab)
    pltpu.sync_copy(at_hbm, at_tab)

    plsc.subcore_barrier()

    n_i = FULL_I + jnp.where(wid < EXTRA, 1, 0)

    def _base(i):
        return pl.multiple_of((i * NW + wid) * EB, EB)

    def _issue_idx(i, q):
        pltpu.async_copy(src_hbm.at[pl.ds(_base(i), EB)], src_buf.at[q],
                         sem_is.at[q])
        pltpu.async_copy(dst_hbm.at[pl.ds(_base(i), EB)], dst_buf.at[q],
                         sem_id.at[q])

    def _wait_idx(i, q):
        pltpu.make_async_copy(src_hbm.at[pl.ds(_base(i), EB)], src_buf.at[q],
                              sem_is.at[q]).wait()
        pltpu.make_async_copy(dst_hbm.at[pl.ds(_base(i), EB)], dst_buf.at[q],
                              sem_id.at[q]).wait()

    def _process(i, q):
        pltpu.async_copy(tv_hbm.at[dst_buf.at[q]], rows, sem).wait()
        for j in range(EB // L):
            si = src_buf[q, pl.ds(j * L, L)]
            di = dst_buf[q, pl.ds(j * L, L)]
            x = plsc.load_gather(ah_tab, [si]) + plsc.load_gather(at_tab, [di])
            x = jnp.clip(x, -10.0, 10.0)
            x = jnp.where(x >= 0.0, x, ALPHA * x)
            w_buf[pl.ds(j * L, L)] = jnp.exp(x)

        def _mbody(g, mcarry):
            wv = w_buf[pl.ds(g * L, L)]
            for l in range(L):
                wr = wv[l]
                r = g * L + l
                for jj in range(D // L):
                    rows[r, pl.ds(jj * L, L)] = rows[r, pl.ds(jj * L, L)] * wr
            return mcarry

        lax.fori_loop(0, EB // L, _mbody, 0)
        pltpu.sync_copy(rows, hp_acc.at[src_buf.at[q]], add=True)
        pltpu.sync_copy(w_buf, rs_acc.at[src_buf.at[q]], add=True)

    _issue_idx(0, 0)

    def _body2(ii, carry):
        for u in range(2):
            i = ii * 2 + u
            _wait_idx(i, u)
            _issue_idx(i + 1, 1 - u)
            _process(i, u)
        return carry

    lax.fori_loop(0, FULL_I // 2, _body2, 0)

    @pl.when(n_i > FULL_I)
    def _():
        _wait_idx(FULL_I, FULL_I % 2)
        _process(FULL_I, FULL_I % 2)

    @pl.when(n_i == FULL_I)
    def _():
        _wait_idx(FULL_I, FULL_I % 2)

    plsc.subcore_barrier()

    pltpu.sync_copy(hp_acc.at[pl.ds(tbase, TILE_ROWS)],
                    hp_out.at[c, pl.ds(tbase, TILE_ROWS)])
    pltpu.sync_copy(rs_acc.at[pl.ds(tbase, TILE_ROWS)],
                    rs_out.at[c, pl.ds(tbase, TILE_ROWS)])


_CB = 1024


def _combine_body(hp_ref, rs_ref, hp_out_ref, rs_out_ref):
    hp_out_ref[...] = hp_ref[0] + hp_ref[1]
    rs_out_ref[...] = (rs_ref[0] + rs_ref[1])[:, None]


def kernel(head_rep, tail_rep, tail_val, edge_list, rel_list, attn):
    f32 = jnp.float32
    i32 = jnp.int32
    head_p = jnp.pad(head_rep.astype(f32), ((0, NPAD - N_NODES), (0, 0)))
    tail_p = jnp.pad(tail_rep.astype(f32), ((0, NPAD - N_NODES), (0, 0)))
    pad_n = E_PAD - N_EDGES
    src_p = jnp.concatenate([edge_list[0].astype(i32),
                             jnp.zeros((pad_n,), dtype=i32)])
    dst_p = jnp.concatenate([edge_list[1].astype(i32),
                             jnp.zeros((pad_n,), dtype=i32)])

    ah2, at2 = pl.pallas_call(
        _scores_body,
        grid=(NPAD // _CB,),
        in_specs=[
            pl.BlockSpec((_CB, D), lambda i: (i, 0)),
            pl.BlockSpec((_CB, D), lambda i: (i, 0)),
            pl.BlockSpec((1, 2 * D), lambda i: (0, 0)),
        ],
        out_specs=[
            pl.BlockSpec((_CB, 1), lambda i: (i, 0)),
            pl.BlockSpec((_CB, 1), lambda i: (i, 0)),
        ],
        out_shape=[
            jax.ShapeDtypeStruct((NPAD, 1), f32),
            jax.ShapeDtypeStruct((NPAD, 1), f32),
        ],
    )(head_p, tail_p, attn.astype(f32))
    ah = ah2.reshape(NPAD)
    at = at2.reshape(NPAD)

    mesh = plsc.VectorSubcoreMesh(core_axis_name="c", subcore_axis_name="s")
    sc_fn = pl.kernel(
        _sc_body,
        mesh=mesh,
        compiler_params=pltpu.CompilerParams(needs_layout_passes=False),
        out_type=[
            jax.ShapeDtypeStruct((NC, NPAD, D), f32),
            jax.ShapeDtypeStruct((NC, NPAD), f32),
        ],
        scratch_types=[
            pltpu.VMEM((NPAD,), f32),
            pltpu.VMEM((NPAD,), f32),
            pltpu.VMEM((2, EB), i32),
            pltpu.VMEM((2, EB), i32),
            pltpu.VMEM((EB,), f32),
            pltpu.VMEM((EB, D), f32),
            pltpu.VMEM_SHARED((NPAD, D), f32),
            pltpu.VMEM_SHARED((NPAD,), f32),
            pltpu.SemaphoreType.DMA,
            pltpu.SemaphoreType.DMA((2,)),
            pltpu.SemaphoreType.DMA((2,)),
        ],
    )
    hp_part, rs_part = sc_fn(ah, at, tail_val.astype(f32), src_p, dst_p)

    hp, rs = pl.pallas_call(
        _combine_body,
        grid=(NPAD // _CB,),
        in_specs=[
            pl.BlockSpec((NC, _CB, D), lambda i: (0, i, 0)),
            pl.BlockSpec((NC, _CB), lambda i: (0, i)),
        ],
        out_specs=[
            pl.BlockSpec((_CB, D), lambda i: (i, 0)),
            pl.BlockSpec((_CB, 1), lambda i: (i, 0)),
        ],
        out_shape=[
            jax.ShapeDtypeStruct((N_NODES, D), f32),
            jax.ShapeDtypeStruct((N_NODES, 1), f32),
        ],
    )(hp_part, rs_part)

    return (rs, hp)


# speedup vs baseline: 2.0095x; 1.0464x over previous
import jax
import jax.numpy as jnp
from jax import lax
from jax.experimental import pallas as pl
from jax.experimental.pallas import tpu as pltpu
from jax.experimental.pallas import tpu_sc as plsc

N_NODES = 10000
NPAD = 10240
N_EDGES = 320000
D = 128
ALPHA = 0.2
NC, NS, L = 2, 16, 16
NW = NC * NS
EB = 128
N_BATCHES = N_EDGES // EB
E_PAD = (N_BATCHES + NW) * EB
TILE_ROWS = NPAD // NS
ROW_CHUNK = 128
N_CHUNKS = TILE_ROWS // ROW_CHUNK
FULL_I = N_BATCHES // NW
EXTRA = N_BATCHES - FULL_I * NW


def _scores_body(head_ref, tail_ref, attn_ref, ah_ref, at_ref):
    aw = attn_ref[...]
    ah_ref[...] = jnp.sum(head_ref[...] * aw[:, :D], axis=1, keepdims=True)
    at_ref[...] = jnp.sum(tail_ref[...] * aw[:, D:], axis=1, keepdims=True)


def _sc_body(ah_hbm, at_hbm, tv_hbm, src_hbm, dst_hbm,
             hp_out, rs_out,
             ah_tab, at_tab, src_buf, dst_buf, w_buf, rows,
             hp_acc, rs_acc, sem, sem_is, sem_id, sem_rs):
    c = lax.axis_index("c")
    s = lax.axis_index("s")
    wid = s * NC + c

    zero16 = jnp.zeros((L,), jnp.float32)

    def _zbody(r, carry):
        for j in range(D // L):
            rows[r, pl.ds(j * L, L)] = zero16
        return carry

    lax.fori_loop(0, ROW_CHUNK, _zbody, 0)

    tbase = s * TILE_ROWS
    for k in range(N_CHUNKS):
        pltpu.sync_copy(rows, hp_acc.at[pl.ds(tbase + k * ROW_CHUNK, ROW_CHUNK)])
        pltpu.sync_copy(rows.at[0], rs_acc.at[pl.ds(tbase + k * ROW_CHUNK, ROW_CHUNK)])

    pltpu.sync_copy(ah_hbm, ah_tab)
    pltpu.sync_copy(at_hbm, at_tab)

    plsc.subcore_barrier()

    n_i = FULL_I + jnp.where(wid < EXTRA, 1, 0)

    def _base(i):
        return pl.multiple_of((i * NW + wid) * EB, EB)

    def _issue_idx(i, q):
        pltpu.async_copy(src_hbm.at[pl.ds(_base(i), EB)], src_buf.at[q],
                         sem_is.at[q])
        pltpu.async_copy(dst_hbm.at[pl.ds(_base(i), EB)], dst_buf.at[q],
                         sem_id.at[q])

    def _wait_idx(i, q):
        pltpu.make_async_copy(src_hbm.at[pl.ds(_base(i), EB)], src_buf.at[q],
                              sem_is.at[q]).wait()
        pltpu.make_async_copy(dst_hbm.at[pl.ds(_base(i), EB)], dst_buf.at[q],
                              sem_id.at[q]).wait()

    def _process(i, q):
        gather = pltpu.async_copy(tv_hbm.at[dst_buf.at[q]], rows, sem)
        for j in range(EB // L):
            si = src_buf[q, pl.ds(j * L, L)]
            di = dst_buf[q, pl.ds(j * L, L)]
            x = plsc.load_gather(ah_tab, [si]) + plsc.load_gather(at_tab, [di])
            x = jnp.clip(x, -10.0, 10.0)
            x = jnp.where(x >= 0.0, x, ALPHA * x)
            w_buf[pl.ds(j * L, L)] = jnp.exp(x)
        gather.wait()

        def _mbody(g, mcarry):
            wv = w_buf[pl.ds(g * L, L)]
            for l in range(L):
                wr = wv[l]
                r = g * L + l
                for jj in range(D // L):
                    rows[r, pl.ds(jj * L, L)] = rows[r, pl.ds(jj * L, L)] * wr
            return mcarry

        lax.fori_loop(0, EB // L, _mbody, 0)
        rs_sc = pltpu.async_copy(w_buf, rs_acc.at[src_buf.at[q]], sem_rs,
                                 add=True)
        pltpu.sync_copy(rows, hp_acc.at[src_buf.at[q]], add=True)
        rs_sc.wait()

    _issue_idx(0, 0)

    def _body2(ii, carry):
        for u in range(2):
            i = ii * 2 + u
            _wait_idx(i, u)
            _issue_idx(i + 1, 1 - u)
            _process(i, u)
        return carry

    lax.fori_loop(0, FULL_I // 2, _body2, 0)

    @pl.when(n_i > FULL_I)
    def _():
        _wait_idx(FULL_I, FULL_I % 2)
        _process(FULL_I, FULL_I % 2)

    @pl.when(n_i == FULL_I)
    def _():
        _wait_idx(FULL_I, FULL_I % 2)

    plsc.subcore_barrier()

    pltpu.sync_copy(hp_acc.at[pl.ds(tbase, TILE_ROWS)],
                    hp_out.at[c, pl.ds(tbase, TILE_ROWS)])
    pltpu.sync_copy(rs_acc.at[pl.ds(tbase, TILE_ROWS)],
                    rs_out.at[c, pl.ds(tbase, TILE_ROWS)])


_CB = 1024


def _combine_body(hp_ref, rs_ref, hp_out_ref, rs_out_ref):
    hp_out_ref[...] = hp_ref[0] + hp_ref[1]
    rs_out_ref[...] = (rs_ref[0] + rs_ref[1])[:, None]


def kernel(head_rep, tail_rep, tail_val, edge_list, rel_list, attn):
    f32 = jnp.float32
    i32 = jnp.int32
    head_p = jnp.pad(head_rep.astype(f32), ((0, NPAD - N_NODES), (0, 0)))
    tail_p = jnp.pad(tail_rep.astype(f32), ((0, NPAD - N_NODES), (0, 0)))
    pad_n = E_PAD - N_EDGES
    src_p = jnp.concatenate([edge_list[0].astype(i32),
                             jnp.zeros((pad_n,), dtype=i32)])
    dst_p = jnp.concatenate([edge_list[1].astype(i32),
                             jnp.zeros((pad_n,), dtype=i32)])

    ah2, at2 = pl.pallas_call(
        _scores_body,
        grid=(NPAD // _CB,),
        in_specs=[
            pl.BlockSpec((_CB, D), lambda i: (i, 0)),
            pl.BlockSpec((_CB, D), lambda i: (i, 0)),
            pl.BlockSpec((1, 2 * D), lambda i: (0, 0)),
        ],
        out_specs=[
            pl.BlockSpec((_CB, 1), lambda i: (i, 0)),
            pl.BlockSpec((_CB, 1), lambda i: (i, 0)),
        ],
        out_shape=[
            jax.ShapeDtypeStruct((NPAD, 1), f32),
            jax.ShapeDtypeStruct((NPAD, 1), f32),
        ],
    )(head_p, tail_p, attn.astype(f32))
    ah = ah2.reshape(NPAD)
    at = at2.reshape(NPAD)

    mesh = plsc.VectorSubcoreMesh(core_axis_name="c", subcore_axis_name="s")
    sc_fn = pl.kernel(
        _sc_body,
        mesh=mesh,
        compiler_params=pltpu.CompilerParams(needs_layout_passes=False),
        out_type=[
            jax.ShapeDtypeStruct((NC, NPAD, D), f32),
            jax.ShapeDtypeStruct((NC, NPAD), f32),
        ],
        scratch_types=[
            pltpu.VMEM((NPAD,), f32),
            pltpu.VMEM((NPAD,), f32),
            pltpu.VMEM((2, EB), i32),
            pltpu.VMEM((2, EB), i32),
            pltpu.VMEM((EB,), f32),
            pltpu.VMEM((EB, D), f32),
            pltpu.VMEM_SHARED((NPAD, D), f32),
            pltpu.VMEM_SHARED((NPAD,), f32),
            pltpu.SemaphoreType.DMA,
            pltpu.SemaphoreType.DMA((2,)),
            pltpu.SemaphoreType.DMA((2,)),
            pltpu.SemaphoreType.DMA,
        ],
    )
    hp_part, rs_part = sc_fn(ah, at, tail_val.astype(f32), src_p, dst_p)

    hp, rs = pl.pallas_call(
        _combine_body,
        grid=(NPAD // _CB,),
        in_specs=[
            pl.BlockSpec((NC, _CB, D), lambda i: (0, i, 0)),
            pl.BlockSpec((NC, _CB), lambda i: (0, i)),
        ],
        out_specs=[
            pl.BlockSpec((_CB, D), lambda i: (i, 0)),
            pl.BlockSpec((_CB, 1), lambda i: (i, 0)),
        ],
        out_shape=[
            jax.ShapeDtypeStruct((N_NODES, D), f32),
            jax.ShapeDtypeStruct((N_NODES, 1), f32),
        ],
    )(hp_part, rs_part)

    return (rs, hp)


# speedup vs baseline: 2.6565x; 1.3220x over previous
import jax
import jax.numpy as jnp
from jax import lax
from jax.experimental import pallas as pl
from jax.experimental.pallas import tpu as pltpu
from jax.experimental.pallas import tpu_sc as plsc

N_NODES = 10000
NPAD = 10240
N_EDGES = 320000
D = 128
ALPHA = 0.2
NC, NS, L = 2, 16, 16
NW = NC * NS
EB = 96
NB = -(-N_EDGES // EB)
E_PAD = 3456 * EB
TILE_ROWS = NPAD // NS
FULL_I = NB // NW
EXTRA = NB - FULL_I * NW
STEADY = 102


def _scores_body(head_ref, tail_ref, attn_ref, ah_ref, at_ref):
    aw = attn_ref[...]
    ah_ref[...] = jnp.sum(head_ref[...] * aw[:, :D], axis=1, keepdims=True)
    at_ref[...] = jnp.sum(tail_ref[...] * aw[:, D:], axis=1, keepdims=True)


def _sc_body(ah_hbm, at_hbm, tv_hbm, src_hbm, dst_hbm,
             hp_out, rs_out,
             ah_tab, at_tab, src_buf, dst_buf, w_buf, rows,
             hp_acc, rs_acc, sem_is, sem_id, sem_g, sem_s, sem_r):
    c = lax.axis_index("c")
    s = lax.axis_index("s")
    wid = s * NC + c

    zero16 = jnp.zeros((L,), jnp.float32)

    def _zbody(r, carry):
        for j in range(D // L):
            rows[0, r, pl.ds(j * L, L)] = zero16
        return carry

    lax.fori_loop(0, EB, _zbody, 0)
    for j in range(EB // L):
        w_buf[0, pl.ds(j * L, L)] = zero16

    tbase = s * TILE_ROWS
    for k in range(TILE_ROWS // EB):
        pltpu.sync_copy(rows.at[0], hp_acc.at[pl.ds(tbase + k * EB, EB)])
        pltpu.sync_copy(w_buf.at[0], rs_acc.at[pl.ds(tbase + k * EB, EB)])
    _rem = TILE_ROWS - (TILE_ROWS // EB) * EB
    pltpu.sync_copy(rows.at[0, pl.ds(0, _rem)],
                    hp_acc.at[pl.ds(tbase + TILE_ROWS - _rem, _rem)])
    pltpu.sync_copy(w_buf.at[0, pl.ds(0, _rem)],
                    rs_acc.at[pl.ds(tbase + TILE_ROWS - _rem, _rem)])

    pltpu.sync_copy(ah_hbm, ah_tab)
    pltpu.sync_copy(at_hbm, at_tab)

    plsc.subcore_barrier()

    has_extra = wid < EXTRA

    def _base(i):
        return pl.multiple_of((i * NW + wid) * EB, 32)

    def _issue_idx(i, e):
        pltpu.async_copy(src_hbm.at[pl.ds(_base(i), EB)], src_buf.at[e],
                         sem_is.at[e])
        pltpu.async_copy(dst_hbm.at[pl.ds(_base(i), EB)], dst_buf.at[e],
                         sem_id.at[e])

    def _wait_idx(i, e):
        pltpu.make_async_copy(src_hbm.at[pl.ds(_base(i), EB)], src_buf.at[e],
                              sem_is.at[e]).wait()
        pltpu.make_async_copy(dst_hbm.at[pl.ds(_base(i), EB)], dst_buf.at[e],
                              sem_id.at[e]).wait()

    def _issue_gather(q, e):
        pltpu.async_copy(tv_hbm.at[dst_buf.at[e]], rows.at[q], sem_g.at[q])

    def _wait_gather(q, e):
        pltpu.make_async_copy(tv_hbm.at[dst_buf.at[e]], rows.at[q],
                              sem_g.at[q]).wait()

    def _issue_scatters(q, e):
        pltpu.async_copy(rows.at[q], hp_acc.at[src_buf.at[e]], sem_s.at[q],
                         add=True)
        pltpu.async_copy(w_buf.at[q], rs_acc.at[src_buf.at[e]], sem_r.at[q],
                         add=True)

    def _wait_scatters(q, e):
        pltpu.make_async_copy(rows.at[q], hp_acc.at[src_buf.at[e]],
                              sem_s.at[q]).wait()
        pltpu.make_async_copy(w_buf.at[q], rs_acc.at[src_buf.at[e]],
                              sem_r.at[q]).wait()

    def _compute(q, e):
        for j in range(EB // L):
            si = src_buf[e, pl.ds(j * L, L)]
            di = dst_buf[e, pl.ds(j * L, L)]
            x = plsc.load_gather(ah_tab, [si]) + plsc.load_gather(at_tab, [di])
            x = jnp.clip(x, -10.0, 10.0)
            x = jnp.where(x >= 0.0, x, ALPHA * x)
            w_buf[q, pl.ds(j * L, L)] = jnp.exp(x)

        def _mbody(g, mcarry):
            wv = w_buf[q, pl.ds(g * L, L)]
            for l in range(L):
                wr = wv[l]
                r = g * L + l
                for jj in range(D // L):
                    rows[q, r, pl.ds(jj * L, L)] = rows[q, r, pl.ds(jj * L, L)] * wr
            return mcarry

        lax.fori_loop(0, EB // L, _mbody, 0)

    _issue_idx(0, 0)
    _issue_idx(1, 1)
    _wait_idx(0, 0)
    _issue_gather(0, 0)

    def _steady(i, q, e, first):
        if not first:
            _wait_scatters(1 - q, (i - 1) % 3)
        _wait_idx(i + 1, (i + 1) % 3)
        _issue_gather(1 - q, (i + 1) % 3)
        _issue_idx(i + 2, (i + 2) % 3)
        _wait_gather(q, e)
        _compute(q, e)
        _issue_scatters(q, e)

    def _body6(ii, carry):
        for u in range(6):
            i = ii * 6 + u

            def _go():
                _steady(i, u % 2, u % 3, False)
            if u == 0:
                @pl.when(ii > 0)
                def _():
                    _go()

                @pl.when(ii == 0)
                def _():
                    _steady(0, 0, 0, True)
            else:
                _go()
        return carry

    lax.fori_loop(0, STEADY // 6, _body6, 0)

    for i in (STEADY, STEADY + 1):
        q = i % 2
        e = i % 3
        _wait_scatters(1 - q, (i - 1) % 3)
        _wait_idx(i + 1, (i + 1) % 3)
        if i == STEADY:
            _issue_gather(1 - q, (i + 1) % 3)
            _issue_idx(i + 2, (i + 2) % 3)
        else:
            @pl.when(has_extra)
            def _():
                _issue_gather(1 - q, (i + 1) % 3)
        _wait_gather(q, e)
        _compute(q, e)
        _issue_scatters(q, e)

    iL = STEADY + 1
    _wait_scatters(iL % 2, iL % 3)

    @pl.when(has_extra)
    def _():
        i = STEADY + 2
        q = i % 2
        e = i % 3
        _wait_gather(q, e)
        _compute(q, e)
        _issue_scatters(q, e)
        _wait_scatters(q, e)

    plsc.subcore_barrier()

    pltpu.sync_copy(hp_acc.at[pl.ds(tbase, TILE_ROWS)],
                    hp_out.at[c, pl.ds(tbase, TILE_ROWS)])
    pltpu.sync_copy(rs_acc.at[pl.ds(tbase, TILE_ROWS)],
                    rs_out.at[c, pl.ds(tbase, TILE_ROWS)])


_CB = 1024


def _combine_body(hp_ref, rs_ref, hp_out_ref, rs_out_ref):
    hp_out_ref[...] = hp_ref[0] + hp_ref[1]
    rs_out_ref[...] = (rs_ref[0] + rs_ref[1])[:, None]


def kernel(head_rep, tail_rep, tail_val, edge_list, rel_list, attn):
    f32 = jnp.float32
    i32 = jnp.int32
    head_p = jnp.pad(head_rep.astype(f32), ((0, NPAD - N_NODES), (0, 0)))
    tail_p = jnp.pad(tail_rep.astype(f32), ((0, NPAD - N_NODES), (0, 0)))
    pad_n = E_PAD - N_EDGES
    pad_src = N_NODES + (jnp.arange(pad_n, dtype=i32) % (NPAD - N_NODES))
    src_p = jnp.concatenate([edge_list[0].astype(i32), pad_src])
    dst_p = jnp.concatenate([edge_list[1].astype(i32),
                             jnp.zeros((pad_n,), dtype=i32)])

    ah2, at2 = pl.pallas_call(
        _scores_body,
        grid=(NPAD // _CB,),
        in_specs=[
            pl.BlockSpec((_CB, D), lambda i: (i, 0)),
            pl.BlockSpec((_CB, D), lambda i: (i, 0)),
            pl.BlockSpec((1, 2 * D), lambda i: (0, 0)),
        ],
        out_specs=[
            pl.BlockSpec((_CB, 1), lambda i: (i, 0)),
            pl.BlockSpec((_CB, 1), lambda i: (i, 0)),
        ],
        out_shape=[
            jax.ShapeDtypeStruct((NPAD, 1), f32),
            jax.ShapeDtypeStruct((NPAD, 1), f32),
        ],
    )(head_p, tail_p, attn.astype(f32))
    ah = ah2.reshape(NPAD)
    at = at2.reshape(NPAD)

    mesh = plsc.VectorSubcoreMesh(core_axis_name="c", subcore_axis_name="s")
    sc_fn = pl.kernel(
        _sc_body,
        mesh=mesh,
        compiler_params=pltpu.CompilerParams(needs_layout_passes=False),
        out_type=[
            jax.ShapeDtypeStruct((NC, NPAD, D), f32),
            jax.ShapeDtypeStruct((NC, NPAD), f32),
        ],
        scratch_types=[
            pltpu.VMEM((NPAD,), f32),
            pltpu.VMEM((NPAD,), f32),
            pltpu.VMEM((3, EB), i32),
            pltpu.VMEM((3, EB), i32),
            pltpu.VMEM((2, EB), f32),
            pltpu.VMEM((2, EB, D), f32),
            pltpu.VMEM_SHARED((NPAD, D), f32),
            pltpu.VMEM_SHARED((NPAD,), f32),
            pltpu.SemaphoreType.DMA((3,)),
            pltpu.SemaphoreType.DMA((3,)),
            pltpu.SemaphoreType.DMA((2,)),
            pltpu.SemaphoreType.DMA((2,)),
            pltpu.SemaphoreType.DMA((2,)),
        ],
    )
    hp_part, rs_part = sc_fn(ah, at, tail_val.astype(f32), src_p, dst_p)

    hp, rs = pl.pallas_call(
        _combine_body,
        grid=(NPAD // _CB,),
        in_specs=[
            pl.BlockSpec((NC, _CB, D), lambda i: (0, i, 0)),
            pl.BlockSpec((NC, _CB), lambda i: (0, i)),
        ],
        out_specs=[
            pl.BlockSpec((_CB, D), lambda i: (i, 0)),
            pl.BlockSpec((_CB, 1), lambda i: (i, 0)),
        ],
        out_shape=[
            jax.ShapeDtypeStruct((N_NODES, D), f32),
            jax.ShapeDtypeStruct((N_NODES, 1), f32),
        ],
    )(hp_part, rs_part)

    return (rs, hp)
